# baseline (speedup 1.0000x reference)
import functools

import jax
import jax.numpy as jnp
from jax import lax
from jax.experimental import pallas as pl
from jax.experimental.pallas import tpu as pltpu
from jax.experimental.pallas import tpu_sc as plsc

N_NODES = 10000
N_PAD = 10240
N_EDGES = 320000
CHUNK = 128
NC, NS = 2, 16
NW = NC * NS
CPT = 80
N_CHUNKS = NW * CPT
NBUF = 2
ISB = 4
ROWS_PER_TILE = N_PAD // NS
D = 128


def _sc_scatter(nfeat, src2d, dst2d):
    mesh = plsc.VectorSubcoreMesh(core_axis_name="c", subcore_axis_name="s")

    @functools.partial(
        pl.kernel,
        out_type=(
            jax.ShapeDtypeStruct((NC, N_PAD, D), jnp.float32),
            jax.ShapeDtypeStruct((NC, N_PAD), jnp.float32),
        ),
        mesh=mesh,
        scratch_types=[
            pltpu.VMEM_SHARED((N_PAD, D), jnp.float32),
            pltpu.VMEM_SHARED((N_PAD,), jnp.float32),
            pltpu.VMEM((CHUNK,), jnp.int32),
            [pltpu.VMEM((CHUNK,), jnp.int32) for _ in range(NBUF)],
            [pltpu.VMEM((CHUNK, D), jnp.float32) for _ in range(NBUF)],
            pltpu.VMEM((CHUNK,), jnp.float32),
            pltpu.VMEM((ROWS_PER_TILE,), jnp.float32),
            [pltpu.SemaphoreType.DMA for _ in range(NBUF)],
        ],
    )
    def k(nfeat_h, src_h, dst_h, out_h, deg_h,
          acc_sh, deg_sh, dst1d, sidx, rows, ones_v, z1d_v,
          gsem):
        c = lax.axis_index("c")
        s = lax.axis_index("s")
        wid = s * NC + c
        chunk0 = wid * CPT

        zeros16 = jnp.zeros((16,), jnp.float32)
        ones16 = jnp.ones((16,), jnp.float32)

        def z1d_body(i, carry):
            z1d_v[pl.ds(i * 16, 16)] = zeros16
            return carry

        lax.fori_loop(0, ROWS_PER_TILE // 16, z1d_body, 0)

        for j in range(CHUNK // 16):
            ones_v[pl.ds(j * 16, 16)] = ones16

        def zrow_body(i, carry):
            for j in range(D // 16):
                rows[0][i, pl.ds(j * 16, 16)] = zeros16
            return carry

        lax.fori_loop(0, CHUNK, zrow_body, 0)

        row0 = s * ROWS_PER_TILE
        for t in range(ROWS_PER_TILE // CHUNK):
            pltpu.sync_copy(rows[0], acc_sh.at[pl.ds(row0 + t * CHUNK, CHUNK)])
        pltpu.sync_copy(z1d_v, deg_sh.at[pl.ds(row0, ROWS_PER_TILE)])

        plsc.subcore_barrier()

        def body(i, carry):
            t = i * NBUF
            pltpu.sync_copy(src_h.at[chunk0 + t], sidx[0])
            pltpu.sync_copy(src_h.at[chunk0 + t + 1], sidx[1])
            g0 = pltpu.async_copy(nfeat_h.at[sidx[0]], rows[0], gsem[0])
            g1 = pltpu.async_copy(nfeat_h.at[sidx[1]], rows[1], gsem[1])
            pltpu.sync_copy(dst_h.at[chunk0 + t], dst1d)
            g0.wait()
            pltpu.sync_copy(rows[0], acc_sh.at[dst1d], add=True)
            pltpu.sync_copy(ones_v, deg_sh.at[dst1d], add=True)
            pltpu.sync_copy(dst_h.at[chunk0 + t + 1], dst1d)
            g1.wait()
            pltpu.sync_copy(rows[1], acc_sh.at[dst1d], add=True)
            pltpu.sync_copy(ones_v, deg_sh.at[dst1d], add=True)
            return carry

        lax.fori_loop(0, CPT // NBUF, body, 0)

        plsc.subcore_barrier()

        pltpu.sync_copy(acc_sh.at[pl.ds(row0, ROWS_PER_TILE)],
                        out_h.at[c, pl.ds(row0, ROWS_PER_TILE)])
        pltpu.sync_copy(deg_sh.at[pl.ds(row0, ROWS_PER_TILE)],
                        deg_h.at[c, pl.ds(row0, ROWS_PER_TILE)])

    return k(nfeat, src2d, dst2d)


def _tc_combine(nfeat, p0, p1, d0, d1, W, b2d):
    BLK = 1000
    grid = (N_NODES // BLK,)

    def body(nf, p0r, p1r, d0r, d1r, wr, br, o):
        d = d0r[...] + d1r[...] + 1.0
        r = 1.0 / d
        sfeat = p0r[...] + p1r[...] + 2.0 * nf[...]
        y = lax.dot_general(sfeat, wr[...], (((1,), (1,)), ((), ())),
                            preferred_element_type=jnp.float32)
        o[...] = y * r + br[...] * (1.0 + r)

    return pl.pallas_call(
        body,
        grid=grid,
        in_specs=[
            pl.BlockSpec((BLK, D), lambda i: (i, 0)),
            pl.BlockSpec((BLK, D), lambda i: (i, 0)),
            pl.BlockSpec((BLK, D), lambda i: (i, 0)),
            pl.BlockSpec((BLK, 1), lambda i: (i, 0)),
            pl.BlockSpec((BLK, 1), lambda i: (i, 0)),
            pl.BlockSpec((D, D), lambda i: (0, 0)),
            pl.BlockSpec((1, D), lambda i: (0, 0)),
        ],
        out_specs=pl.BlockSpec((BLK, D), lambda i: (i, 0)),
        out_shape=jax.ShapeDtypeStruct((N_NODES, D), jnp.float32),
    )(nfeat, p0, p1, d0, d1, W, b2d)


def kernel(nfeat, edge_index, W_neigh, b_neigh):
    src = edge_index[0].astype(jnp.int32)
    dst = edge_index[1].astype(jnp.int32)
    pad = N_CHUNKS * CHUNK - N_EDGES
    src2d = jnp.concatenate(
        [src, jnp.zeros((pad,), jnp.int32)]).reshape(N_CHUNKS, CHUNK)
    dst2d = jnp.concatenate(
        [dst, jnp.full((pad,), N_NODES, jnp.int32)]).reshape(N_CHUNKS, CHUNK)
    partial, deg = _sc_scatter(nfeat, src2d, dst2d)
    out = _tc_combine(nfeat, partial[0], partial[1],
                      deg[0].reshape(N_PAD, 1), deg[1].reshape(N_PAD, 1),
                      W_neigh, b_neigh.reshape(1, D))
    return out


# speedup vs baseline: 4.2014x; 4.2014x over previous
import functools

import jax
import jax.numpy as jnp
from jax import lax
from jax.experimental import pallas as pl
from jax.experimental.pallas import tpu as pltpu
from jax.experimental.pallas import tpu_sc as plsc

N_NODES = 10000
N_PAD = 10240
N_EDGES = 320000
CHUNK = 128
NC, NS = 2, 16
NW = NC * NS
CPT = 80
N_CHUNKS = NW * CPT
NBUF = 2
ISB = 4
ROWS_PER_TILE = N_PAD // NS
D = 128


def _sc_scatter(nfeat, src2d, dst2d):
    mesh = plsc.VectorSubcoreMesh(core_axis_name="c", subcore_axis_name="s")

    @functools.partial(
        pl.kernel,
        out_type=(
            jax.ShapeDtypeStruct((NC, N_PAD, D), jnp.float32),
            jax.ShapeDtypeStruct((NC, N_PAD), jnp.float32),
        ),
        mesh=mesh,
        scratch_types=[
            pltpu.VMEM_SHARED((N_PAD, D), jnp.float32),
            pltpu.VMEM_SHARED((N_PAD,), jnp.float32),
            pltpu.VMEM((CPT, CHUNK), jnp.int32),
            [pltpu.VMEM((CHUNK,), jnp.int32) for _ in range(ISB)],
            [pltpu.VMEM((CHUNK, D), jnp.float32) for _ in range(NBUF)],
            pltpu.VMEM((CHUNK,), jnp.float32),
            pltpu.VMEM((ROWS_PER_TILE,), jnp.float32),
            [pltpu.SemaphoreType.DMA for _ in range(NBUF)],
            [pltpu.SemaphoreType.DMA for _ in range(NBUF)],
            [pltpu.SemaphoreType.DMA for _ in range(ISB)],
            [pltpu.SemaphoreType.DMA for _ in range(ISB)],
        ],
    )
    def k(nfeat_h, src_h, dst_h, out_h, deg_h,
          acc_sh, deg_sh, dst_v, sidx, rows, ones_v, z1d_v,
          gsem, ssem, dsem, isem):
        c = lax.axis_index("c")
        s = lax.axis_index("s")
        wid = s * NC + c
        chunk0 = wid * CPT

        zeros16 = jnp.zeros((16,), jnp.float32)
        ones16 = jnp.ones((16,), jnp.float32)

        def z1d_body(i, carry):
            z1d_v[pl.ds(i * 16, 16)] = zeros16
            return carry

        lax.fori_loop(0, ROWS_PER_TILE // 16, z1d_body, 0)

        for j in range(CHUNK // 16):
            ones_v[pl.ds(j * 16, 16)] = ones16

        def zrow_body(i, carry):
            for j in range(D // 16):
                rows[0][i, pl.ds(j * 16, 16)] = zeros16
            return carry

        lax.fori_loop(0, CHUNK, zrow_body, 0)

        row0 = s * ROWS_PER_TILE
        for t in range(ROWS_PER_TILE // CHUNK):
            pltpu.sync_copy(rows[0], acc_sh.at[pl.ds(row0 + t * CHUNK, CHUNK)])
        pltpu.sync_copy(z1d_v, deg_sh.at[pl.ds(row0, ROWS_PER_TILE)])

        pltpu.sync_copy(dst_h.at[pl.ds(chunk0, CPT)], dst_v)
        plsc.subcore_barrier()

        for q in range(ISB):
            pltpu.async_copy(src_h.at[chunk0 + q], sidx[q], isem[q])
        for b in range(NBUF):
            pltpu.make_async_copy(src_h.at[chunk0 + b], sidx[b],
                                  isem[b]).wait()
            pltpu.async_copy(nfeat_h.at[sidx[b]], rows[b], gsem[b])

        def body(i, carry):
            for u in range(ISB):
                t = i * ISB + u
                b = u % NBUF
                q2 = (u + NBUF) % ISB

                pltpu.make_async_copy(
                    nfeat_h.at[sidx[u]], rows[b], gsem[b]).wait()

                sc = pltpu.async_copy(
                    rows[b], acc_sh.at[dst_v.at[t]], ssem[b], add=True)

                @pl.when(t >= ISB)
                def _():
                    pltpu.make_async_copy(
                        ones_v, deg_sh.at[dst_v.at[t - ISB]], dsem[u]).wait()

                pltpu.async_copy(ones_v, deg_sh.at[dst_v.at[t]], dsem[u])

                sc.wait()

                @pl.when(t + ISB < CPT)
                def _():
                    pltpu.async_copy(src_h.at[chunk0 + t + ISB],
                                     sidx[u], isem[u])

                @pl.when(t + NBUF < CPT)
                def _():
                    pltpu.make_async_copy(src_h.at[chunk0 + t + NBUF],
                                          sidx[q2], isem[q2]).wait()
                    pltpu.async_copy(nfeat_h.at[sidx[q2]], rows[b], gsem[b])
            return carry

        lax.fori_loop(0, CPT // ISB, body, 0)

        for u in range(ISB):
            t = CPT - ISB + u
            pltpu.make_async_copy(
                ones_v, deg_sh.at[dst_v.at[t]], dsem[u]).wait()

        plsc.subcore_barrier()

        pltpu.sync_copy(acc_sh.at[pl.ds(row0, ROWS_PER_TILE)],
                        out_h.at[c, pl.ds(row0, ROWS_PER_TILE)])
        pltpu.sync_copy(deg_sh.at[pl.ds(row0, ROWS_PER_TILE)],
                        deg_h.at[c, pl.ds(row0, ROWS_PER_TILE)])

    return k(nfeat, src2d, dst2d)


def _tc_combine(nfeat, p0, p1, d0, d1, W, b2d):
    BLK = 1000
    grid = (N_NODES // BLK,)

    def body(nf, p0r, p1r, d0r, d1r, wr, br, o):
        d = d0r[...] + d1r[...] + 1.0
        r = 1.0 / d
        sfeat = p0r[...] + p1r[...] + 2.0 * nf[...]
        y = lax.dot_general(sfeat, wr[...], (((1,), (1,)), ((), ())),
                            preferred_element_type=jnp.float32)
        o[...] = y * r + br[...] * (1.0 + r)

    return pl.pallas_call(
        body,
        grid=grid,
        in_specs=[
            pl.BlockSpec((BLK, D), lambda i: (i, 0)),
            pl.BlockSpec((BLK, D), lambda i: (i, 0)),
            pl.BlockSpec((BLK, D), lambda i: (i, 0)),
            pl.BlockSpec((BLK, 1), lambda i: (i, 0)),
            pl.BlockSpec((BLK, 1), lambda i: (i, 0)),
            pl.BlockSpec((D, D), lambda i: (0, 0)),
            pl.BlockSpec((1, D), lambda i: (0, 0)),
        ],
        out_specs=pl.BlockSpec((BLK, D), lambda i: (i, 0)),
        out_shape=jax.ShapeDtypeStruct((N_NODES, D), jnp.float32),
    )(nfeat, p0, p1, d0, d1, W, b2d)


def kernel(nfeat, edge_index, W_neigh, b_neigh):
    src = edge_index[0].astype(jnp.int32)
    dst = edge_index[1].astype(jnp.int32)
    pad = N_CHUNKS * CHUNK - N_EDGES
    ar = jnp.arange(pad, dtype=jnp.int32)
    src2d = jnp.concatenate(
        [src, ar % N_NODES]).reshape(N_CHUNKS, CHUNK)
    dst2d = jnp.concatenate(
        [dst, N_NODES + ar % (N_PAD - N_NODES)]).reshape(N_CHUNKS, CHUNK)
    partial, deg = _sc_scatter(nfeat, src2d, dst2d)
    out = _tc_combine(nfeat, partial[0], partial[1],
                      deg[0].reshape(N_PAD, 1), deg[1].reshape(N_PAD, 1),
                      W_neigh, b_neigh.reshape(1, D))
    return out


# speedup vs baseline: 4.2219x; 1.0049x over previous
import functools

import jax
import jax.numpy as jnp
from jax import lax
from jax.experimental import pallas as pl
from jax.experimental.pallas import tpu as pltpu
from jax.experimental.pallas import tpu_sc as plsc

N_NODES = 10000
N_PAD = 10240
N_EDGES = 320000
CHUNK = 128
NC, NS = 2, 16
NW = NC * NS
CPT = 80
N_CHUNKS = NW * CPT
NBUF = 2
ISB = 4
ROWS_PER_TILE = N_PAD // NS
D = 128


def _sc_scatter(nfeat, src2d, dst2d):
    mesh = plsc.VectorSubcoreMesh(core_axis_name="c", subcore_axis_name="s")

    @functools.partial(
        pl.kernel,
        out_type=(
            jax.ShapeDtypeStruct((NC, N_PAD, D), jnp.float32),
            jax.ShapeDtypeStruct((NC, N_PAD), jnp.float32),
        ),
        mesh=mesh,
        scratch_types=[
            pltpu.VMEM_SHARED((N_PAD, D), jnp.float32),
            pltpu.VMEM_SHARED((N_PAD,), jnp.float32),
            [pltpu.VMEM((CHUNK,), jnp.int32) for _ in range(ISB)],
            [pltpu.VMEM((CHUNK,), jnp.int32) for _ in range(ISB)],
            [pltpu.VMEM((CHUNK, D), jnp.float32) for _ in range(NBUF)],
            pltpu.VMEM((CHUNK,), jnp.float32),
            pltpu.VMEM((ROWS_PER_TILE,), jnp.float32),
            [pltpu.SemaphoreType.DMA for _ in range(NBUF)],
            [pltpu.SemaphoreType.DMA for _ in range(NBUF)],
            [pltpu.SemaphoreType.DMA for _ in range(ISB)],
            [pltpu.SemaphoreType.DMA for _ in range(ISB)],
            [pltpu.SemaphoreType.DMA for _ in range(ISB)],
        ],
    )
    def k(nfeat_h, src_h, dst_h, out_h, deg_h,
          acc_sh, deg_sh, didx, sidx, rows, ones_v, z1d_v,
          gsem, ssem, dsem, isem, jsem):
        c = lax.axis_index("c")
        s = lax.axis_index("s")
        wid = s * NC + c
        chunk0 = wid * CPT

        zeros16 = jnp.zeros((16,), jnp.float32)
        ones16 = jnp.ones((16,), jnp.float32)

        def z1d_body(i, carry):
            z1d_v[pl.ds(i * 16, 16)] = zeros16
            return carry

        lax.fori_loop(0, ROWS_PER_TILE // 16, z1d_body, 0)

        for j in range(CHUNK // 16):
            ones_v[pl.ds(j * 16, 16)] = ones16

        def zrow_body(i, carry):
            for j in range(D // 16):
                rows[0][i, pl.ds(j * 16, 16)] = zeros16
            return carry

        lax.fori_loop(0, CHUNK, zrow_body, 0)

        row0 = s * ROWS_PER_TILE
        for t in range(ROWS_PER_TILE // CHUNK):
            pltpu.sync_copy(rows[0], acc_sh.at[pl.ds(row0 + t * CHUNK, CHUNK)])
        pltpu.sync_copy(z1d_v, deg_sh.at[pl.ds(row0, ROWS_PER_TILE)])

        plsc.subcore_barrier()

        for q in range(ISB):
            pltpu.async_copy(src_h.at[chunk0 + q], sidx[q], isem[q])
            pltpu.async_copy(dst_h.at[chunk0 + q], didx[q], jsem[q])
        for b in range(NBUF):
            pltpu.make_async_copy(src_h.at[chunk0 + b], sidx[b],
                                  isem[b]).wait()
            pltpu.async_copy(nfeat_h.at[sidx[b]], rows[b], gsem[b])

        def body(i, carry):
            for u in range(ISB):
                t = i * ISB + u
                b = u % NBUF
                q2 = (u + NBUF) % ISB

                pltpu.make_async_copy(
                    nfeat_h.at[sidx[u]], rows[b], gsem[b]).wait()
                pltpu.make_async_copy(
                    dst_h.at[chunk0 + t], didx[u], jsem[u]).wait()

                sc = pltpu.async_copy(
                    rows[b], acc_sh.at[didx[u]], ssem[b], add=True)
                dg = pltpu.async_copy(ones_v, deg_sh.at[didx[u]], dsem[u])
                sc.wait()
                dg.wait()

                @pl.when(t + ISB < CPT)
                def _():
                    pltpu.async_copy(src_h.at[chunk0 + t + ISB],
                                     sidx[u], isem[u])
                    pltpu.async_copy(dst_h.at[chunk0 + t + ISB],
                                     didx[u], jsem[u])

                @pl.when(t + NBUF < CPT)
                def _():
                    pltpu.make_async_copy(src_h.at[chunk0 + t + NBUF],
                                          sidx[q2], isem[q2]).wait()
                    pltpu.async_copy(nfeat_h.at[sidx[q2]], rows[b], gsem[b])
            return carry

        lax.fori_loop(0, CPT // ISB, body, 0)

        plsc.subcore_barrier()

        pltpu.sync_copy(acc_sh.at[pl.ds(row0, ROWS_PER_TILE)],
                        out_h.at[c, pl.ds(row0, ROWS_PER_TILE)])
        pltpu.sync_copy(deg_sh.at[pl.ds(row0, ROWS_PER_TILE)],
                        deg_h.at[c, pl.ds(row0, ROWS_PER_TILE)])

    return k(nfeat, src2d, dst2d)


def _tc_combine(nfeat, p0, p1, d0, d1, W, b2d):
    BLK = 1000
    grid = (N_NODES // BLK,)

    def body(nf, p0r, p1r, d0r, d1r, wr, br, o):
        d = d0r[...] + d1r[...] + 1.0
        r = 1.0 / d
        sfeat = p0r[...] + p1r[...] + 2.0 * nf[...]
        y = lax.dot_general(sfeat, wr[...], (((1,), (1,)), ((), ())),
                            preferred_element_type=jnp.float32)
        o[...] = y * r + br[...] * (1.0 + r)

    return pl.pallas_call(
        body,
        grid=grid,
        in_specs=[
            pl.BlockSpec((BLK, D), lambda i: (i, 0)),
            pl.BlockSpec((BLK, D), lambda i: (i, 0)),
            pl.BlockSpec((BLK, D), lambda i: (i, 0)),
            pl.BlockSpec((BLK, 1), lambda i: (i, 0)),
            pl.BlockSpec((BLK, 1), lambda i: (i, 0)),
            pl.BlockSpec((D, D), lambda i: (0, 0)),
            pl.BlockSpec((1, D), lambda i: (0, 0)),
        ],
        out_specs=pl.BlockSpec((BLK, D), lambda i: (i, 0)),
        out_shape=jax.ShapeDtypeStruct((N_NODES, D), jnp.float32),
    )(nfeat, p0, p1, d0, d1, W, b2d)


def kernel(nfeat, edge_index, W_neigh, b_neigh):
    src = edge_index[0].astype(jnp.int32)
    dst = edge_index[1].astype(jnp.int32)
    pad = N_CHUNKS * CHUNK - N_EDGES
    ar = jnp.arange(pad, dtype=jnp.int32)
    src2d = jnp.concatenate(
        [src, ar % N_NODES]).reshape(N_CHUNKS, CHUNK)
    dst2d = jnp.concatenate(
        [dst, N_NODES + ar % (N_PAD - N_NODES)]).reshape(N_CHUNKS, CHUNK)
    partial, deg = _sc_scatter(nfeat, src2d, dst2d)
    out = _tc_combine(nfeat, partial[0], partial[1],
                      deg[0].reshape(N_PAD, 1), deg[1].reshape(N_PAD, 1),
                      W_neigh, b_neigh.reshape(1, D))
    return out
